# baseline (speedup 1.0000x reference)
import functools
import jax
import jax.numpy as jnp
from jax import lax
from jax.experimental import pallas as pl
from jax.experimental.pallas import tpu as pltpu
from jax.experimental.pallas import tpu_sc as plsc

N_USER = 5000
N_NODES = 10000
NP = 10240
D = 128
DH = D // 2
E = 320000
L = 16
NC = 2
NS = 16
E_PER_TILE = E // NS
B = 80
NBLK = E_PER_TILE // B
CHUNKI = 125
NCHUNK = NBLK // CHUNKI
NQUAD = (CHUNKI - 2) // 4
DJ = DH // L
ROWS_PER_TILE = NP // NS
ZCHUNKS = ROWS_PER_TILE // B
NSLOT = 6
PD = 3

_mesh = plsc.VectorSubcoreMesh(
    core_axis_name="c", subcore_axis_name="s", num_cores=NC, num_subcores=NS)


@functools.partial(
    pl.kernel,
    out_type=jax.ShapeDtypeStruct((NC, NP, DH), jnp.float32),
    mesh=_mesh,
    scratch_types=[
        pltpu.VMEM((CHUNKI, B), jnp.int32),
        pltpu.VMEM((CHUNKI, B), jnp.int32),
        pltpu.VMEM((CHUNKI, B), jnp.float32),
        [pltpu.VMEM((B, DH), jnp.float32)] * NSLOT,
        pltpu.VMEM_SHARED((NP, DH), jnp.float32),
        [pltpu.SemaphoreType.DMA] * NSLOT,
        [pltpu.SemaphoreType.DMA] * NSLOT,
    ],
    compiler_params=pltpu.CompilerParams(
        needs_layout_passes=False, use_tc_tiling_on_sc=False),
)
def _accumulate(x_hbm, dst_hbm, src_hbm, w_hbm, out_hbm,
                didx2, sidx2, wbuf2, rowbufs, acc, gsems, ssems):
    cid = lax.axis_index("c")
    sid = lax.axis_index("s")

    zeros = jnp.zeros((L,), jnp.float32)

    @pl.loop(0, B)
    def _zero(i):
        for j in range(DJ):
            rowbufs[0][i, pl.ds(j * L, L)] = zeros

    for k in range(ZCHUNKS):
        r0 = sid * ROWS_PER_TILE + k * B
        pltpu.sync_copy(rowbufs[0], acc.at[pl.ds(r0, B)])
    plsc.subcore_barrier()

    xc = x_hbm.at[cid]

    def issue_gather(j, s):
        pltpu.async_copy(xc.at[didx2.at[j]], rowbufs[s], gsems[s])

    def wait_gather(s):
        pltpu.make_async_copy(xc.at[pl.ds(0, B)], rowbufs[s], gsems[s]).wait()

    def issue_scatter(j, s):
        pltpu.async_copy(rowbufs[s], acc.at[sidx2.at[j]], ssems[s], add=True)

    def wait_scatter(s):
        pltpu.make_async_copy(xc.at[pl.ds(0, B)], rowbufs[s], ssems[s]).wait()

    def step(b, s, prefetch, wait_prev_scatter=True):
        if prefetch:
            s_pre = (s + PD) % NSLOT
            if wait_prev_scatter:
                wait_scatter(s_pre)
            issue_gather(b + PD, s_pre)
        wait_gather(s)
        scale_only(b, s)
        issue_scatter(b, s)

    def scale_only(j, s):
        rows = rowbufs[s]

        @plsc.parallel_loop(0, B, 1, unroll=8)
        def _edge(e):
            wvec = wbuf2[j, pl.ds((e // L) * L, L)]
            wsp = lax.gather(
                wvec, jnp.full((L, 1), e % L, jnp.int32),
                lax.GatherDimensionNumbers(
                    offset_dims=(), collapsed_slice_dims=(0,),
                    start_index_map=(0,)),
                (1,), mode=lax.GatherScatterMode.PROMISE_IN_BOUNDS)
            for k in range(DJ):
                rows[e, pl.ds(k * L, L)] = rows[e, pl.ds(k * L, L)] * wsp

    @pl.loop(0, NCHUNK)
    def _chunk(c):
        pltpu.sync_copy(dst_hbm.at[sid, c], didx2)
        pltpu.sync_copy(src_hbm.at[sid, c], sidx2)
        pltpu.sync_copy(w_hbm.at[sid, c], wbuf2)

        for s in range(PD):
            issue_gather(s, s)

        for b in range(NSLOT):
            step(b, b, prefetch=True, wait_prev_scatter=(b >= PD))

        @pl.loop(1, (CHUNKI - NSLOT - 5) // NSLOT + 1)
        def _sextet(q):
            b0 = NSLOT * q
            for i in range(NSLOT):
                step(b0 + i, i, prefetch=True)

        for b in range(CHUNKI - 5, CHUNKI):
            step(b, b % NSLOT, prefetch=(b + PD < CHUNKI))
        for s in range(NSLOT):
            wait_scatter(s)

    plsc.subcore_barrier()

    for k in range(ZCHUNKS):
        r0 = sid * ROWS_PER_TILE + k * B
        pltpu.sync_copy(acc.at[pl.ds(r0, B)], out_hbm.at[cid, pl.ds(r0, B)])


@jax.jit
def kernel(ini_embeds, edge_index, adj_values):
    src = edge_index[0].astype(jnp.int32).reshape(NS, NCHUNK, CHUNKI, B)
    dst = edge_index[1].astype(jnp.int32).reshape(NS, NCHUNK, CHUNKI, B)
    w = adj_values.reshape(NS, NCHUNK, CHUNKI, B)

    x0 = jnp.stack([ini_embeds[:, :DH], ini_embeds[:, DH:]])

    o1 = _accumulate(x0, dst, src, w)
    o2 = _accumulate(o1, dst, src, w)

    tem = jnp.concatenate(
        [o1[0, :N_NODES], o1[1, :N_NODES], o2[0, :N_NODES], o2[1, :N_NODES]],
        axis=-1)
    return tem[:N_USER], tem[N_USER:]


# speedup vs baseline: 1.0062x; 1.0062x over previous
import functools
import jax
import jax.numpy as jnp
from jax import lax
from jax.experimental import pallas as pl
from jax.experimental.pallas import tpu as pltpu
from jax.experimental.pallas import tpu_sc as plsc

N_USER = 5000
N_NODES = 10000
NP = 10240
D = 128
DH = D // 2
E = 320000
L = 16
NC = 2
NS = 16
E_PER_TILE = E // NS
B = 80
NBLK = E_PER_TILE // B
CHUNKI = 125
NCHUNK = NBLK // CHUNKI
NQUAD = (CHUNKI - 2) // 4
DJ = DH // L
ROWS_PER_TILE = NP // NS
ZCHUNKS = ROWS_PER_TILE // B
NSLOT = 4

_mesh = plsc.VectorSubcoreMesh(
    core_axis_name="c", subcore_axis_name="s", num_cores=NC, num_subcores=NS)


@functools.partial(
    pl.kernel,
    out_type=jax.ShapeDtypeStruct((NC, NP, DH), jnp.float32),
    mesh=_mesh,
    scratch_types=[
        pltpu.VMEM((CHUNKI, B), jnp.int32),
        pltpu.VMEM((CHUNKI, B), jnp.int32),
        pltpu.VMEM((CHUNKI, B), jnp.float32),
        [pltpu.VMEM((B, DH), jnp.float32)] * NSLOT,
        pltpu.VMEM_SHARED((NP, DH), jnp.float32),
        [pltpu.SemaphoreType.DMA] * NSLOT,
        [pltpu.SemaphoreType.DMA] * NSLOT,
    ],
    compiler_params=pltpu.CompilerParams(
        needs_layout_passes=False, use_tc_tiling_on_sc=False),
)
def _accumulate(x_hbm, dst_hbm, src_hbm, w_hbm, out_hbm,
                didx2, sidx2, wbuf2, rowbufs, acc, gsems, ssems):
    cid = lax.axis_index("c")
    sid = lax.axis_index("s")

    zeros = jnp.zeros((L,), jnp.float32)

    @pl.loop(0, B)
    def _zero(i):
        for j in range(DJ):
            rowbufs[0][i, pl.ds(j * L, L)] = zeros

    for k in range(ZCHUNKS):
        r0 = sid * ROWS_PER_TILE + k * B
        pltpu.sync_copy(rowbufs[0], acc.at[pl.ds(r0, B)])
    plsc.subcore_barrier()

    xc = x_hbm.at[cid]

    def issue_gather(j, s):
        pltpu.async_copy(xc.at[didx2.at[j]], rowbufs[s], gsems[s])

    def wait_gather(s):
        pltpu.make_async_copy(xc.at[pl.ds(0, B)], rowbufs[s], gsems[s]).wait()

    def issue_scatter(j, s):
        pltpu.async_copy(rowbufs[s], acc.at[sidx2.at[j]], ssems[s], add=True)

    def wait_scatter(s):
        pltpu.make_async_copy(xc.at[pl.ds(0, B)], rowbufs[s], ssems[s]).wait()

    def step(b, s, prefetch, wait_prev_scatter=True):
        if prefetch:
            s_pre = (s + 2) % NSLOT
            if wait_prev_scatter:
                wait_scatter(s_pre)
            issue_gather(b + 2, s_pre)
        wait_gather(s)
        scale_only(b, s)
        issue_scatter(b, s)

    def scale_only(j, s):
        rows = rowbufs[s]

        @plsc.parallel_loop(0, B, 1, unroll=8)
        def _edge(e):
            wvec = wbuf2[j, pl.ds((e // L) * L, L)]
            wsp = lax.gather(
                wvec, jnp.full((L, 1), e % L, jnp.int32),
                lax.GatherDimensionNumbers(
                    offset_dims=(), collapsed_slice_dims=(0,),
                    start_index_map=(0,)),
                (1,), mode=lax.GatherScatterMode.PROMISE_IN_BOUNDS)
            for k in range(DJ):
                rows[e, pl.ds(k * L, L)] = rows[e, pl.ds(k * L, L)] * wsp

    @pl.loop(0, NCHUNK)
    def _chunk(c):
        pltpu.sync_copy(dst_hbm.at[sid, c], didx2)
        pltpu.sync_copy(src_hbm.at[sid, c], sidx2)
        pltpu.sync_copy(w_hbm.at[sid, c], wbuf2)

        issue_gather(0, 0)
        issue_gather(1, 1)

        step(0, 0, prefetch=True, wait_prev_scatter=False)
        step(1, 1, prefetch=True, wait_prev_scatter=False)
        step(2, 2, prefetch=True)
        step(3, 3, prefetch=True)

        @pl.loop(1, NQUAD)
        def _quad(q):
            b0 = 4 * q
            for i in range(4):
                step(b0 + i, i, prefetch=True)

        for b in range(4 * NQUAD, CHUNKI):
            step(b, b % NSLOT, prefetch=(b + 2 < CHUNKI))
        for s in range(NSLOT):
            wait_scatter(s)

    plsc.subcore_barrier()

    for k in range(ZCHUNKS):
        r0 = sid * ROWS_PER_TILE + k * B
        pltpu.sync_copy(acc.at[pl.ds(r0, B)], out_hbm.at[cid, pl.ds(r0, B)])


@jax.jit
def kernel(ini_embeds, edge_index, adj_values):
    src = edge_index[0].astype(jnp.int32).reshape(NS, NCHUNK, CHUNKI, B)
    dst = edge_index[1].astype(jnp.int32).reshape(NS, NCHUNK, CHUNKI, B)
    w = adj_values.reshape(NS, NCHUNK, CHUNKI, B)

    x0 = jnp.stack([ini_embeds[:, :DH], ini_embeds[:, DH:]])

    o1 = _accumulate(x0, dst, src, w)
    o2 = _accumulate(o1, dst, src, w)

    h1 = jnp.concatenate([o1[0, :N_NODES], o1[1, :N_NODES]], axis=-1)
    h2 = jnp.concatenate([o2[0, :N_NODES], o2[1, :N_NODES]], axis=-1)
    tem = jnp.concatenate([h1, h2], axis=-1)
    return tem[:N_USER], tem[N_USER:]
